# Initial kernel scaffold; baseline (speedup 1.0000x reference)
import jax
import jax.numpy as jnp
from jax.experimental import pallas as pl


def kernel(x, edge_index, adj_values, W):
    raise NotImplementedError("write your pallas kernel here")


# speedup vs baseline: 6.7533x; 6.7533x over previous
import functools

import jax
import jax.numpy as jnp
from jax import lax
from jax.experimental import pallas as pl
from jax.experimental.pallas import tpu as pltpu
from jax.experimental.pallas import tpu_sc as plsc

N = 10000
E = 320000
D = 128
NC = 2
NS = 16
NW = NC * NS
EPW = E // NW
C = 80
R = EPW // C
NP = 10240
RPT = NP // NS
ZR = 80
LANES = 16


def _sc_body(x_hbm, col_hbm, row_hbm, adj_hbm, zero_hbm, out_hbm,
             col_v, row_v, adj_v, rows_v, acc_sh, sem, sem2):
    c = lax.axis_index("c")
    s = lax.axis_index("s")
    wid = s * NC + c

    pltpu.sync_copy(col_hbm.at[wid], col_v)
    pltpu.sync_copy(row_hbm.at[wid], row_v)
    pltpu.sync_copy(zero_hbm, rows_v)
    for k in range(RPT // ZR):
        pltpu.sync_copy(rows_v, acc_sh.at[pl.ds(s * RPT + k * ZR, ZR)])
    plsc.subcore_barrier()

    def round_body(r, carry):
        acp = pltpu.async_copy(
            adj_hbm.at[pl.ds(wid * EPW + r * C, C)], adj_v, sem2)
        gcp = pltpu.async_copy(x_hbm.at[col_v.at[r]], rows_v, sem)
        acp.wait()
        gcp.wait()

        dn = lax.GatherDimensionNumbers(
            offset_dims=(), collapsed_slice_dims=(0,), start_index_map=(0,))

        def scale_group(g, _):
            av = adj_v[pl.ds(g * LANES, LANES)]
            base = g * LANES
            for e in range(LANES):
                a = lax.gather(av, jnp.full((LANES, 1), e, jnp.int32), dn,
                               (1,), mode=lax.GatherScatterMode.PROMISE_IN_BOUNDS)
                for j in range(D // LANES):
                    sl = pl.ds(j * LANES, LANES)
                    rows_v[base + e, sl] = rows_v[base + e, sl] * a
            return _

        lax.fori_loop(0, C // LANES, scale_group, 0)

        pltpu.sync_copy(rows_v, acc_sh.at[row_v.at[r]], add=True)
        return carry

    lax.fori_loop(0, R, round_body, 0)
    plsc.subcore_barrier()

    for k in range(RPT // ZR):
        off = s * RPT + k * ZR
        pltpu.sync_copy(acc_sh.at[pl.ds(off, ZR)], rows_v)
        pltpu.sync_copy(rows_v, out_hbm.at[c, pl.ds(off, ZR)])


_sc_scatter = pl.kernel(
    _sc_body,
    mesh=plsc.VectorSubcoreMesh(core_axis_name="c", subcore_axis_name="s"),
    out_type=jax.ShapeDtypeStruct((NC, NP, D), jnp.float32),
    scratch_types=[
        pltpu.VMEM((NW * R // NW, C), jnp.int32),
        pltpu.VMEM((NW * R // NW, C), jnp.int32),
        pltpu.VMEM((C,), jnp.float32),
        pltpu.VMEM((C, D), jnp.float32),
        pltpu.VMEM_SHARED((NP, D), jnp.float32),
        pltpu.SemaphoreType.DMA,
        pltpu.SemaphoreType.DMA,
    ],
)


def _mm_body(p_ref, w_ref, o_ref):
    o_ref[...] = jnp.dot(p_ref[0] + p_ref[1], w_ref[...],
                         preferred_element_type=jnp.float32)


BM = 1000


@functools.partial(jax.jit, donate_argnums=())
def kernel(x, edge_index, adj_values, W):
    col2 = edge_index[1].astype(jnp.int32).reshape(NW, R, C)
    row2 = edge_index[0].astype(jnp.int32).reshape(NW, R, C)
    zeros = jnp.zeros((ZR, D), jnp.float32)
    partial = _sc_scatter(x, col2, row2, adj_values, zeros)
    out = pl.pallas_call(
        _mm_body,
        grid=(N // BM,),
        in_specs=[
            pl.BlockSpec((NC, BM, D), lambda i: (0, i, 0)),
            pl.BlockSpec((D, D), lambda i: (0, 0)),
        ],
        out_specs=pl.BlockSpec((BM, D), lambda i: (i, 0)),
        out_shape=jax.ShapeDtypeStruct((N, D), jnp.float32),
    )(partial, W)
    return out


# speedup vs baseline: 10.1975x; 1.5100x over previous
import functools

import jax
import jax.numpy as jnp
from jax import lax
from jax.experimental import pallas as pl
from jax.experimental.pallas import tpu as pltpu
from jax.experimental.pallas import tpu_sc as plsc

N = 10000
E = 320000
D = 128
NC = 2
NS = 16
NW = NC * NS
EPW = E // NW
C = 80
R = EPW // C
NP = 10240
RPT = NP // NS
ZR = 80
LANES = 16

_DN = lax.GatherDimensionNumbers(
    offset_dims=(), collapsed_slice_dims=(0,), start_index_map=(0,))


def _sc_body(x_hbm, col_hbm, row_hbm, adj_hbm, zero_hbm, out_hbm,
             col_v, row_b, adj_v, rows_v, acc_sh, sem0, sem1,
             asem0, asem1, rsem0, rsem1):
    c = lax.axis_index("c")
    s = lax.axis_index("s")
    wid = s * NC + c

    pltpu.sync_copy(col_hbm.at[wid], col_v)

    pltpu.sync_copy(zero_hbm, rows_v.at[0])
    for k in range(RPT // ZR):
        pltpu.sync_copy(rows_v.at[0], acc_sh.at[pl.ds(s * RPT + k * ZR, ZR)])
    plsc.subcore_barrier()

    sems = (sem0, sem1)
    asems = (asem0, asem1)
    rsems = (rsem0, rsem1)

    def gather_round(r, b):
        pltpu.async_copy(
            adj_hbm.at[pl.ds(wid * EPW + r * C, C)], adj_v.at[b], asems[b])
        pltpu.async_copy(
            row_hbm.at[pl.ds(wid * EPW + r * C, C)], row_b.at[b], rsems[b])
        pltpu.async_copy(x_hbm.at[col_v.at[r]], rows_v.at[b], sems[b])

    def drain_round(b):
        pltpu.make_async_copy(adj_hbm.at[pl.ds(0, C)], adj_v.at[b],
                              asems[b]).wait()
        pltpu.make_async_copy(row_hbm.at[pl.ds(0, C)], row_b.at[b],
                              rsems[b]).wait()
        pltpu.make_async_copy(x_hbm.at[pl.ds(0, C), :], rows_v.at[b],
                              sems[b]).wait()

    def process_round(r, b):
        rb = rows_v.at[b]

        def scale_group(g, _):
            av = adj_v[b, pl.ds(g * LANES, LANES)]
            base = g * LANES
            for e in range(LANES):
                a = lax.gather(av, jnp.full((LANES, 1), e, jnp.int32), _DN,
                               (1,), mode=lax.GatherScatterMode.PROMISE_IN_BOUNDS)
                for j in range(D // LANES):
                    sl = pl.ds(j * LANES, LANES)
                    rb[base + e, sl] = rb[base + e, sl] * a
            return _

        lax.fori_loop(0, C // LANES, scale_group, 0)

        pltpu.sync_copy(rb, acc_sh.at[row_b.at[b]], add=True)

    gather_round(0, 0)

    def pipe_body(r2, carry):
        for b in range(2):
            r = r2 * 2 + b
            drain_round(b)
            gather_round(lax.rem(r + 1, R), 1 - b)
            process_round(r, b)
        return carry

    lax.fori_loop(0, R // 2, pipe_body, 0)
    drain_round(0)
    if R % 2 == 1:
        process_round(R - 1, 0)
    plsc.subcore_barrier()

    for k in range(RPT // ZR):
        off = s * RPT + k * ZR
        pltpu.sync_copy(acc_sh.at[pl.ds(off, ZR)], rows_v.at[0])
        pltpu.sync_copy(rows_v.at[0], out_hbm.at[c, pl.ds(off, ZR)])


_sc_scatter = pl.kernel(
    _sc_body,
    mesh=plsc.VectorSubcoreMesh(core_axis_name="c", subcore_axis_name="s"),
    out_type=jax.ShapeDtypeStruct((NC, NP, D), jnp.float32),
    scratch_types=[
        pltpu.VMEM((R, C), jnp.int32),
        pltpu.VMEM((2, C), jnp.int32),
        pltpu.VMEM((2, C), jnp.float32),
        pltpu.VMEM((2, C, D), jnp.float32),
        pltpu.VMEM_SHARED((NP, D), jnp.float32),
        pltpu.SemaphoreType.DMA,
        pltpu.SemaphoreType.DMA,
        pltpu.SemaphoreType.DMA,
        pltpu.SemaphoreType.DMA,
        pltpu.SemaphoreType.DMA,
        pltpu.SemaphoreType.DMA,
    ],
)


def _mm_body(p_ref, w_ref, o_ref):
    o_ref[...] = jnp.dot(p_ref[0] + p_ref[1], w_ref[...],
                         preferred_element_type=jnp.float32)


BM = 1000


@functools.partial(jax.jit, donate_argnums=())
def kernel(x, edge_index, adj_values, W):
    col2 = edge_index[1].astype(jnp.int32).reshape(NW, R, C)
    row2 = edge_index[0].astype(jnp.int32)
    zeros = jnp.zeros((ZR, D), jnp.float32)
    partial = _sc_scatter(x, col2, row2, adj_values, zeros)
    out = pl.pallas_call(
        _mm_body,
        grid=(N // BM,),
        in_specs=[
            pl.BlockSpec((NC, BM, D), lambda i: (0, i, 0)),
            pl.BlockSpec((D, D), lambda i: (0, 0)),
        ],
        out_specs=pl.BlockSpec((BM, D), lambda i: (i, 0)),
        out_shape=jax.ShapeDtypeStruct((N, D), jnp.float32),
    )(partial, W)
    return out
